# baseline (speedup 1.0000x reference)
import functools

import jax
import jax.numpy as jnp
import numpy as np
from jax.experimental import pallas as pl
from jax.experimental.pallas import tpu as pltpu

B, T, D, K = 4096, 64, 256, 16
BB = 128
SB = 128
NSTEP = B // BB

_Q_LO = 22
_Q_FRAC = np.float32(0.35 * (T - 1) - _Q_LO)
_NEG = np.float32(np.finfo(np.float32).min)


def _body(ao_ref, no_ref, f_ref, proj_ref, out_ref, tk_s, nm_s):
    i = pl.program_id(0)

    for s in range(BB // SB):
        f = f_ref[s * SB:(s + 1) * SB].reshape(SB * T, D)
        scores = jax.lax.dot_general(
            f, proj_ref[...], (((1,), (0,)), ((), ())),
            preferred_element_type=jnp.float32)
        s3 = scores.reshape(SB, T, K)

        ao = ao_ref[s * SB:(s + 1) * SB]
        le = (ao[:, :, None] <= ao[:, None, :]).astype(jnp.float32)
        cnt = jnp.sum(le, axis=1)
        v_lo = jnp.min(jnp.where(cnt >= _Q_LO + 1, ao, jnp.inf), axis=1)
        v_hi = jnp.min(jnp.where(cnt >= _Q_LO + 2, ao, jnp.inf), axis=1)
        th = v_lo + _Q_FRAC * (v_hi - v_lo)

        masked = jnp.where(ao >= th[:, None], ao, _NEG)
        m3 = masked[:, :, None]

        kio = jax.lax.broadcasted_iota(jnp.int32, (SB, T, K), 2)
        idx = jnp.argmax(s3, axis=2, keepdims=True)
        val3 = jnp.where(kio == idx, m3, _NEG)

        cmax = jnp.max(val3, axis=1)
        present = cmax > _NEG
        vsum = jnp.sum(jnp.where(present, cmax, 0.0), axis=1)
        ncl = jnp.sum(present.astype(jnp.float32), axis=1)
        tk_s[0, pl.ds(i * BB + s * SB, SB)] = vsum / jnp.maximum(ncl, 1.0)
        nm_s[pl.ds(i * BB + s * SB, SB), :] = (
            jnp.max(no_ref[s * SB:(s + 1) * SB], axis=1)[:, None])

    @pl.when(i == NSTEP - 1)
    def _hinge():
        c = 1.0 - tk_s[...]

        def chunk(j, acc):
            nmj = nm_s[pl.ds(j * 512, 512), :]
            return acc + jnp.sum(jnp.maximum(nmj + c, 0.0))

        acc = jax.lax.fori_loop(0, B // 512, chunk, jnp.float32(0.0))
        out_ref[0, 0] = acc / np.float32(B)


@jax.jit
def _run(ao, no, feats, proj):
    out = pl.pallas_call(
        _body,
        grid=(NSTEP,),
        in_specs=[
            pl.BlockSpec((BB, T), lambda i: (i, 0)),
            pl.BlockSpec((BB, T), lambda i: (i, 0)),
            pl.BlockSpec((BB, T, D), lambda i: (i, 0, 0)),
            pl.BlockSpec((D, K), lambda i: (0, 0)),
        ],
        out_specs=pl.BlockSpec(memory_space=pltpu.SMEM),
        out_shape=jax.ShapeDtypeStruct((1, 1), jnp.float32),
        scratch_shapes=[
            pltpu.VMEM((1, B), jnp.float32),
            pltpu.VMEM((B, 1), jnp.float32),
        ],
        compiler_params=pltpu.CompilerParams(
            vmem_limit_bytes=120 * 1024 * 1024),
    )(ao, no, feats, proj)
    return out[0, 0]


def kernel(abnormal_outputs, normal_outputs, abnormal_features,
           normal_features, proj, sim_th, out_th):
    del normal_features, sim_th, out_th
    return _run(abnormal_outputs, normal_outputs, abnormal_features, proj)


# speedup vs baseline: 1.0324x; 1.0324x over previous
import functools

import jax
import jax.numpy as jnp
import numpy as np
from jax.experimental import pallas as pl
from jax.experimental.pallas import tpu as pltpu

B, T, D, K = 4096, 64, 256, 16
BB = 256
NSTEP = B // BB

_Q_LO = 22
_Q_FRAC = np.float32(0.35 * (T - 1) - _Q_LO)
_NEG = np.float32(np.finfo(np.float32).min)


def _body(ao_ref, no_ref, f_ref, proj_ref, out_ref, tk_s, nm_s):
    i = pl.program_id(0)

    f = f_ref[...].reshape(BB * T, D)
    scores = jax.lax.dot_general(
        f, proj_ref[...], (((1,), (0,)), ((), ())),
        preferred_element_type=jnp.float32)
    s3 = scores.reshape(BB, T, K)

    ao = ao_ref[...]
    le = (ao[:, :, None] <= ao[:, None, :]).astype(jnp.float32)
    cnt = jnp.sum(le, axis=1)
    v_lo = jnp.min(jnp.where(cnt >= _Q_LO + 1, ao, jnp.inf), axis=1)
    v_hi = jnp.min(jnp.where(cnt >= _Q_LO + 2, ao, jnp.inf), axis=1)
    th = v_lo + _Q_FRAC * (v_hi - v_lo)

    masked = jnp.where(ao >= th[:, None], ao, _NEG)
    m3 = masked[:, :, None]

    kio = jax.lax.broadcasted_iota(jnp.int32, (BB, T, K), 2)
    idx = jnp.argmax(s3, axis=2, keepdims=True)
    val3 = jnp.where(kio == idx, m3, _NEG)

    cmax = jnp.max(val3, axis=1)
    present = cmax > _NEG
    vsum = jnp.sum(jnp.where(present, cmax, 0.0), axis=1)
    ncl = jnp.sum(present.astype(jnp.float32), axis=1)
    tk_s[0, pl.ds(i * BB, BB)] = vsum / jnp.maximum(ncl, 1.0)
    nm_s[pl.ds(i * BB, BB), :] = jnp.max(no_ref[...], axis=1)[:, None]

    @pl.when(i == NSTEP - 1)
    def _hinge():
        c = 1.0 - tk_s[...]

        def chunk(j, acc):
            nmj = nm_s[pl.ds(j * 512, 512), :]
            return acc + jnp.sum(jnp.maximum(nmj + c, 0.0))

        acc = jax.lax.fori_loop(0, B // 512, chunk, jnp.float32(0.0))
        out_ref[0, 0] = acc / np.float32(B)


@jax.jit
def _run(ao, no, feats, proj):
    out = pl.pallas_call(
        _body,
        grid=(NSTEP,),
        in_specs=[
            pl.BlockSpec((BB, T), lambda i: (i, 0)),
            pl.BlockSpec((BB, T), lambda i: (i, 0)),
            pl.BlockSpec((BB, T, D), lambda i: (i, 0, 0)),
            pl.BlockSpec((D, K), lambda i: (0, 0)),
        ],
        out_specs=pl.BlockSpec(memory_space=pltpu.SMEM),
        out_shape=jax.ShapeDtypeStruct((1, 1), jnp.float32),
        scratch_shapes=[
            pltpu.VMEM((1, B), jnp.float32),
            pltpu.VMEM((B, 1), jnp.float32),
        ],
        compiler_params=pltpu.CompilerParams(
            vmem_limit_bytes=120 * 1024 * 1024),
    )(ao, no, feats, proj)
    return out[0, 0]


def kernel(abnormal_outputs, normal_outputs, abnormal_features,
           normal_features, proj, sim_th, out_th):
    del normal_features, sim_th, out_th
    return _run(abnormal_outputs, normal_outputs, abnormal_features, proj)


# speedup vs baseline: 1.0334x; 1.0010x over previous
import jax
import jax.numpy as jnp
import numpy as np
from jax.experimental import pallas as pl
from jax.experimental.pallas import tpu as pltpu

B, T, D, K = 4096, 64, 256, 16
BB = 256
NSTEP = B // BB

_Q_LO = 22
_Q_FRAC = np.float32(0.35 * (T - 1) - _Q_LO)
_NEG = np.float32(np.finfo(np.float32).min)


def _body(ao_ref, no_ref, f_ref, proj_ref, out_ref, tk_s, nm_s):
    i = pl.program_id(0)

    f = f_ref[...].reshape(BB * T, D)
    scores = jax.lax.dot_general(
        f, proj_ref[...], (((1,), (0,)), ((), ())),
        preferred_element_type=jnp.float32)
    s3 = scores.reshape(BB, T, K)

    ao = ao_ref[...]
    le = (ao[:, :, None] <= ao[:, None, :]).astype(jnp.float32)
    cnt = jnp.sum(le, axis=1)
    v_lo = jnp.min(jnp.where(cnt >= _Q_LO + 1, ao, jnp.inf), axis=1)
    v_hi = jnp.min(jnp.where(cnt >= _Q_LO + 2, ao, jnp.inf), axis=1)
    th = v_lo + _Q_FRAC * (v_hi - v_lo)

    masked = jnp.where(ao >= th[:, None], ao, _NEG)
    m3 = masked[:, :, None]

    kio = jax.lax.broadcasted_iota(jnp.int32, (BB, T, K), 2)
    idx = jnp.argmax(s3, axis=2, keepdims=True)
    val3 = jnp.where(kio == idx, m3, _NEG)

    cmax = jnp.max(val3, axis=1)
    present = cmax > _NEG
    vsum = jnp.sum(jnp.where(present, cmax, 0.0), axis=1)
    ncl = jnp.sum(present.astype(jnp.float32), axis=1)
    tk_s[0, pl.ds(i * BB, BB)] = vsum / jnp.maximum(ncl, 1.0)
    nm_s[pl.ds(i * BB, BB), :] = jnp.max(no_ref[...], axis=1)[:, None]

    @pl.when(i == NSTEP - 1)
    def _hinge():
        c = 1.0 - tk_s[...]

        def chunk(j, acc):
            nmj = nm_s[pl.ds(j * 512, 512), :]
            return acc + jnp.sum(jnp.maximum(nmj + c, 0.0))

        acc = jax.lax.fori_loop(0, B // 512, chunk, jnp.float32(0.0))
        out_ref[0, 0] = acc / np.float32(B)


@jax.jit
def _run(ao, no, feats, proj):
    out = pl.pallas_call(
        _body,
        grid=(NSTEP,),
        in_specs=[
            pl.BlockSpec((BB, T), lambda i: (i, 0)),
            pl.BlockSpec((BB, T), lambda i: (i, 0)),
            pl.BlockSpec((BB, T, D), lambda i: (i, 0, 0)),
            pl.BlockSpec((D, K), lambda i: (0, 0)),
        ],
        out_specs=pl.BlockSpec(memory_space=pltpu.SMEM),
        out_shape=jax.ShapeDtypeStruct((1, 1), jnp.float32),
        scratch_shapes=[
            pltpu.VMEM((1, B), jnp.float32),
            pltpu.VMEM((B, 1), jnp.float32),
        ],
        compiler_params=pltpu.CompilerParams(
            vmem_limit_bytes=120 * 1024 * 1024),
    )(ao, no, feats, proj)
    return out[0, 0]


def kernel(abnormal_outputs, normal_outputs, abnormal_features,
           normal_features, proj, sim_th, out_th):
    del normal_features, sim_th, out_th
    return _run(abnormal_outputs, normal_outputs, abnormal_features, proj)
